# baseline (speedup 1.0000x reference)
import jax
import jax.numpy as jnp
from jax import lax
from jax.experimental import pallas as pl
from jax.experimental.pallas import tpu as pltpu
from jax.experimental.pallas import tpu_sc as plsc

VOCAB = 1000000
DIM = 64
EPS = 1e-05
L = 16
NC = 2
NS = 16
NW = NC * NS
B_TOTAL = 16384 * 50
NSPLIT = 1
B_CALL = B_TOTAL // NSPLIT
W_ROWS = B_CALL // NW
CH = 256
NCH = W_ROWS // CH
SUB = CH // 128
NB = CH // L
IB = W_ROWS // 128


def _ntz(n):
    b = 0
    while not (n >> b) & 1:
        b += 1
    return b


def _rsqrt(x):
    i = plsc.bitcast(x, jnp.int32)
    i = jnp.full((L,), 0x5F3759DF, jnp.int32) - lax.shift_right_logical(
        i, jnp.full((L,), 1, jnp.int32))
    y = plsc.bitcast(i, jnp.float32)
    half = jnp.full((L,), 0.5, jnp.float32)
    three_half = jnp.full((L,), 1.5, jnp.float32)
    for _ in range(3):
        y = y * (three_half - half * x * y * y)
    return y


def _sc_body(table_hbm, idx_hbm, gam_hbm, bet_hbm, out_hbm,
             idx_v, rows_v, norm_v, out_v, gam_v, bet_v, sem_g, sem_w):
    wid = lax.axis_index("s") * NC + lax.axis_index("c")
    pltpu.sync_copy(gam_hbm, gam_v)
    pltpu.sync_copy(bet_hbm, bet_v)
    g_regs = [gam_v[pl.ds(L * k, L)] for k in range(4)]
    b_regs = [bet_v[pl.ds(L * k, L)] for k in range(4)]
    iota = lax.iota(jnp.int32, L)
    const65 = iota * jnp.full((L,), DIM + 1, jnp.int32)
    zeros16 = jnp.zeros((L,), jnp.int32)
    inv_d = jnp.full((L,), 1.0 / DIM, jnp.float32)
    zero = jnp.zeros((L,), jnp.float32)
    eps = jnp.full((L,), EPS, jnp.float32)

    def stage_idx(g):
        pltpu.sync_copy(
            idx_hbm.at[pl.ds(wid * IB + g * SUB, SUB)],
            idx_v.at[g % 2])

    def start_gather(g):
        for j in range(SUB):
            pltpu.async_copy(table_hbm.at[idx_v.at[g % 2, j]],
                             rows_v.at[g % 2, pl.ds(128 * j, 128)], sem_g)

    def wait_gather(g):
        for j in range(SUB):
            pltpu.make_async_copy(table_hbm.at[idx_v.at[g % 2, j]],
                                  rows_v.at[g % 2, pl.ds(128 * j, 128)],
                                  sem_g).wait()

    def start_wb(g):
        pltpu.async_copy(out_v.at[g % 2],
                         out_hbm.at[pl.ds(wid * W_ROWS + g * CH, CH)], sem_w)

    def wait_wb(g):
        pltpu.make_async_copy(out_v.at[g % 2],
                              out_hbm.at[pl.ds(wid * W_ROWS + g * CH, CH)],
                              sem_w).wait()

    stage_idx(0)
    start_gather(0)

    @pl.loop(0, NCH)
    def _chunk(g):
        @pl.when(g < NCH - 1)
        def _prefetch():
            stage_idx(g + 1)

        wait_gather(g)

        @pl.when(g < NCH - 1)
        def _launch():
            start_gather(g + 1)

        @pl.when(g >= 1)
        def _drain():
            wait_wb(g - 1)

        rbuf = rows_v.at[g % 2]
        obuf = out_v.at[g % 2]

        @pl.loop(0, NB, unroll=4)
        def _block(b):
            flat = const65 + b * (L * DIM)
            accs = [zero, zero, zero, zero]
            accs2 = [zero, zero, zero, zero]
            for grp in range(DIM // 8):
                flats_g = []
                for j in range(8):
                    d = grp * 8 + j
                    flats_g.append(flat)
                    if d < DIM - 1:
                        flat = flat ^ (1 << _ntz(d + 1))
                vals = [plsc.load_gather(rbuf, [zeros16, f]) for f in flats_g]
                for i, v in enumerate(vals):
                    accs[i % 4] = accs[i % 4] + v
                    accs2[i % 4] = accs2[i % 4] + v * v
            acc = (accs[0] + accs[1]) + (accs[2] + accs[3])
            acc2 = (accs2[0] + accs2[1]) + (accs2[2] + accs2[3])
            mean = acc * inv_d
            var = acc2 * inv_d - mean * mean
            rstd = _rsqrt(jnp.maximum(var, zero) + eps)
            flat = const65 + b * (L * DIM)
            for grp in range(DIM // 8):
                flats_g = []
                for j in range(8):
                    d = grp * 8 + j
                    flats_g.append(flat)
                    if d < DIM - 1:
                        flat = flat ^ (1 << (5 - _ntz(d + 1)))
                vals = [plsc.load_gather(rbuf, [zeros16, f]) for f in flats_g]
                outs = [(v - mean) * rstd for v in vals]
                for f, o in zip(flats_g, outs):
                    plsc.store_scatter(norm_v, [zeros16, f], o)
            for r in range(0, L, 2):
                sls = [(b * L + r + rr, pl.ds(L * k, L))
                       for rr in range(2) for k in range(4)]
                vs = [norm_v[sl] for sl in sls]
                os_ = [v * g_regs[i % 4] + b_regs[i % 4]
                       for i, v in enumerate(vs)]
                for sl, o in zip(sls, os_):
                    obuf[sl] = o

        start_wb(g)

    wait_wb(NCH - 1)


@jax.jit
def _run(table, idx2d, gamma, beta):
    mesh = plsc.VectorSubcoreMesh(core_axis_name="c", subcore_axis_name="s",
                                  num_cores=NC, num_subcores=NS)
    f = pl.kernel(
        _sc_body,
        out_type=jax.ShapeDtypeStruct((B_CALL, DIM), jnp.float32),
        mesh=mesh,
        scratch_types=[
            pltpu.VMEM((2, SUB, 128), jnp.int32),
            pltpu.VMEM((2, CH, DIM), jnp.float32),
            pltpu.VMEM((CH, DIM), jnp.float32),
            pltpu.VMEM((2, CH, DIM), jnp.float32),
            pltpu.VMEM((DIM,), jnp.float32),
            pltpu.VMEM((DIM,), jnp.float32),
            pltpu.SemaphoreType.DMA,
            pltpu.SemaphoreType.DMA,
        ],
        compiler_params=pltpu.CompilerParams(needs_layout_passes=False,
                                             use_tc_tiling_on_sc=False),
    )
    return f(table, idx2d, gamma, beta)


def kernel(input_ids, table, gamma, beta):
    bsz, seq = input_ids.shape
    idx2d = input_ids.astype(jnp.int32).reshape(-1, 128)
    nrow = idx2d.shape[0] // NSPLIT
    outs = [_run(table, idx2d[i * nrow:(i + 1) * nrow], gamma, beta)
            for i in range(NSPLIT)]
    out = jnp.concatenate(outs, axis=0)
    return out.reshape(bsz, seq, DIM)


# speedup vs baseline: 1.4027x; 1.4027x over previous
import jax
import jax.numpy as jnp
from jax import lax
from jax.experimental import pallas as pl
from jax.experimental.pallas import tpu as pltpu
from jax.experimental.pallas import tpu_sc as plsc

VOCAB = 1000000
DIM = 64
EPS = 1e-05
L = 16
NC = 2
NS = 16
NW = NC * NS
B_TOTAL = 16384 * 50
NSPLIT = 1
B_CALL = B_TOTAL // NSPLIT
W_ROWS = B_CALL // NW
CH = 256
NCH = W_ROWS // CH
SUB = CH // 128
NB = CH // L
IB = W_ROWS // 128


def _ntz(n):
    b = 0
    while not (n >> b) & 1:
        b += 1
    return b


def _rsqrt(x):
    i = plsc.bitcast(x, jnp.int32)
    i = jnp.full((L,), 0x5F3759DF, jnp.int32) - lax.shift_right_logical(
        i, jnp.full((L,), 1, jnp.int32))
    y = plsc.bitcast(i, jnp.float32)
    half = jnp.full((L,), 0.5, jnp.float32)
    three_half = jnp.full((L,), 1.5, jnp.float32)
    for _ in range(3):
        y = y * (three_half - half * x * y * y)
    return y


def _sc_body(table_hbm, idx_hbm, gam_hbm, bet_hbm, out_hbm,
             idx_v, rows_v, norm_v, out_v, gam_v, bet_v, sem_g, sem_w):
    wid = lax.axis_index("s") * NC + lax.axis_index("c")
    pltpu.sync_copy(gam_hbm, gam_v)
    pltpu.sync_copy(bet_hbm, bet_v)
    g_regs = [gam_v[pl.ds(L * k, L)] for k in range(4)]
    b_regs = [bet_v[pl.ds(L * k, L)] for k in range(4)]
    iota = lax.iota(jnp.int32, L)
    const65 = iota * jnp.full((L,), DIM + 1, jnp.int32)
    zeros16 = jnp.zeros((L,), jnp.int32)
    inv_d = jnp.full((L,), 1.0 / DIM, jnp.float32)
    zero = jnp.zeros((L,), jnp.float32)
    eps = jnp.full((L,), EPS, jnp.float32)

    def stage_idx(g):
        pltpu.sync_copy(
            idx_hbm.at[pl.ds(wid * IB + g * SUB, SUB)],
            idx_v.at[g % 2])

    def start_gather(g):
        for j in range(SUB):
            pltpu.async_copy(table_hbm.at[idx_v.at[g % 2, j]],
                             rows_v.at[g % 2, pl.ds(128 * j, 128)], sem_g)

    def wait_gather(g):
        for j in range(SUB):
            pltpu.make_async_copy(table_hbm.at[idx_v.at[g % 2, j]],
                                  rows_v.at[g % 2, pl.ds(128 * j, 128)],
                                  sem_g).wait()

    def start_wb(g):
        pltpu.async_copy(out_v.at[g % 2],
                         out_hbm.at[pl.ds(wid * W_ROWS + g * CH, CH)], sem_w)

    def wait_wb(g):
        pltpu.make_async_copy(out_v.at[g % 2],
                              out_hbm.at[pl.ds(wid * W_ROWS + g * CH, CH)],
                              sem_w).wait()

    stage_idx(0)
    start_gather(0)

    @pl.loop(0, NCH)
    def _chunk(g):
        @pl.when(g < NCH - 1)
        def _prefetch():
            stage_idx(g + 1)

        wait_gather(g)

        @pl.when(g < NCH - 1)
        def _launch():
            start_gather(g + 1)

        @pl.when(g >= 1)
        def _drain():
            wait_wb(g - 1)

        rbuf = rows_v.at[g % 2]
        obuf = out_v.at[g % 2]

        @pl.loop(0, NB, unroll=2)
        def _block(b):
            for r0 in range(0, L, 8):
                rows_g = [b * L + (r0 + rr) for rr in range(8)]
                vs = [[rbuf[q, pl.ds(L * k, L)] for k in range(4)]
                      for q in rows_g]
                stats = []
                for v in vs:
                    s = (v[0] + v[1]) + (v[2] + v[3])
                    t = (v[0] * v[0] + v[1] * v[1]) + (v[2] * v[2] + v[3] * v[3])
                    stats.append((plsc.cumsum(s), plsc.cumsum(t)))
                for q, v, (cs, ct) in zip(rows_g, vs, stats):
                    mean_r = cs[15] * (1.0 / DIM)
                    var_r = ct[15] * (1.0 / DIM) - mean_r * mean_r
                    x = jnp.maximum(var_r, 0.0) + EPS
                    i = lax.bitcast_convert_type(x, jnp.int32)
                    i = 0x5F3759DF - lax.shift_right_logical(i, 1)
                    y = lax.bitcast_convert_type(i, jnp.float32)
                    for _ in range(3):
                        y = y * (1.5 - 0.5 * x * y * y)
                    m = zero + mean_r
                    rs = zero + y
                    for k in range(4):
                        obuf[q, pl.ds(L * k, L)] = (
                            (v[k] - m) * rs * g_regs[k] + b_regs[k])

        start_wb(g)

    wait_wb(NCH - 1)


@jax.jit
def _run(table, idx2d, gamma, beta):
    mesh = plsc.VectorSubcoreMesh(core_axis_name="c", subcore_axis_name="s",
                                  num_cores=NC, num_subcores=NS)
    f = pl.kernel(
        _sc_body,
        out_type=jax.ShapeDtypeStruct((B_CALL, DIM), jnp.float32),
        mesh=mesh,
        scratch_types=[
            pltpu.VMEM((2, SUB, 128), jnp.int32),
            pltpu.VMEM((2, CH, DIM), jnp.float32),
            pltpu.VMEM((CH, DIM), jnp.float32),
            pltpu.VMEM((2, CH, DIM), jnp.float32),
            pltpu.VMEM((DIM,), jnp.float32),
            pltpu.VMEM((DIM,), jnp.float32),
            pltpu.SemaphoreType.DMA,
            pltpu.SemaphoreType.DMA,
        ],
        compiler_params=pltpu.CompilerParams(needs_layout_passes=False,
                                             use_tc_tiling_on_sc=False),
    )
    return f(table, idx2d, gamma, beta)


def kernel(input_ids, table, gamma, beta):
    bsz, seq = input_ids.shape
    idx2d = input_ids.astype(jnp.int32).reshape(-1, 128)
    nrow = idx2d.shape[0] // NSPLIT
    outs = [_run(table, idx2d[i * nrow:(i + 1) * nrow], gamma, beta)
            for i in range(NSPLIT)]
    out = jnp.concatenate(outs, axis=0)
    return out.reshape(bsz, seq, DIM)


# speedup vs baseline: 1.4085x; 1.0041x over previous
import jax
import jax.numpy as jnp
from jax import lax
from jax.experimental import pallas as pl
from jax.experimental.pallas import tpu as pltpu
from jax.experimental.pallas import tpu_sc as plsc

VOCAB = 1000000
DIM = 64
EPS = 1e-05
L = 16
NC = 2
NS = 16
NW = NC * NS
B_TOTAL = 16384 * 50
NSPLIT = 1
B_CALL = B_TOTAL // NSPLIT
W_ROWS = B_CALL // NW
CH = 256
NCH = W_ROWS // CH
SUB = CH // 128
NB = CH // L
IB = W_ROWS // 128


def _ntz(n):
    b = 0
    while not (n >> b) & 1:
        b += 1
    return b


def _rsqrt(x):
    i = plsc.bitcast(x, jnp.int32)
    i = jnp.full((L,), 0x5F3759DF, jnp.int32) - lax.shift_right_logical(
        i, jnp.full((L,), 1, jnp.int32))
    y = plsc.bitcast(i, jnp.float32)
    half = jnp.full((L,), 0.5, jnp.float32)
    three_half = jnp.full((L,), 1.5, jnp.float32)
    for _ in range(3):
        y = y * (three_half - half * x * y * y)
    return y


def _sc_body(table_hbm, idx_hbm, gam_hbm, bet_hbm, out_hbm,
             idx_v, rows_v, norm_v, out_v, gam_v, bet_v, sem_g, sem_w):
    wid = lax.axis_index("s") * NC + lax.axis_index("c")
    pltpu.sync_copy(gam_hbm, gam_v)
    pltpu.sync_copy(bet_hbm, bet_v)
    g_regs = [gam_v[pl.ds(L * k, L)] for k in range(4)]
    b_regs = [bet_v[pl.ds(L * k, L)] for k in range(4)]
    iota = lax.iota(jnp.int32, L)
    const65 = iota * jnp.full((L,), DIM + 1, jnp.int32)
    zeros16 = jnp.zeros((L,), jnp.int32)
    inv_d = jnp.full((L,), 1.0 / DIM, jnp.float32)
    zero = jnp.zeros((L,), jnp.float32)
    eps = jnp.full((L,), EPS, jnp.float32)

    def stage_idx(g):
        pltpu.sync_copy(
            idx_hbm.at[pl.ds(wid * IB + g * SUB, SUB)],
            idx_v.at[g % 2])

    def start_gather(g):
        for j in range(SUB):
            pltpu.async_copy(table_hbm.at[idx_v.at[g % 2, j]],
                             rows_v.at[g % 2, pl.ds(128 * j, 128)], sem_g)

    def wait_gather(g):
        for j in range(SUB):
            pltpu.make_async_copy(table_hbm.at[idx_v.at[g % 2, j]],
                                  rows_v.at[g % 2, pl.ds(128 * j, 128)],
                                  sem_g).wait()

    def start_wb(g):
        pltpu.async_copy(out_v.at[g % 2],
                         out_hbm.at[pl.ds(wid * W_ROWS + g * CH, CH)], sem_w)

    def wait_wb(g):
        pltpu.make_async_copy(out_v.at[g % 2],
                              out_hbm.at[pl.ds(wid * W_ROWS + g * CH, CH)],
                              sem_w).wait()

    stage_idx(0)
    start_gather(0)

    @pl.loop(0, NCH)
    def _chunk(g):
        @pl.when(g < NCH - 1)
        def _prefetch():
            stage_idx(g + 1)

        wait_gather(g)

        @pl.when(g < NCH - 1)
        def _launch():
            start_gather(g + 1)

        @pl.when(g >= 1)
        def _drain():
            wait_wb(g - 1)

        rbuf = rows_v.at[g % 2]
        obuf = out_v.at[g % 2]

        @pl.loop(0, NB)
        def _block(b):
            for r0 in range(0, L, 8):
                rows_g = [b * L + (r0 + rr) for rr in range(8)]
                vs = [[rbuf[q, pl.ds(L * k, L)] for k in range(4)]
                      for q in rows_g]
                stats = []
                for v in vs:
                    s = (v[0] + v[1]) + (v[2] + v[3])
                    t = (v[0] * v[0] + v[1] * v[1]) + (v[2] * v[2] + v[3] * v[3])
                    stats.append((plsc.cumsum(s), plsc.cumsum(t)))
                for q, v, (cs, ct) in zip(rows_g, vs, stats):
                    mean_r = cs[15] * (1.0 / DIM)
                    var_r = ct[15] * (1.0 / DIM) - mean_r * mean_r
                    x = jnp.maximum(var_r, 0.0) + EPS
                    i = lax.bitcast_convert_type(x, jnp.int32)
                    i = 0x5F3759DF - lax.shift_right_logical(i, 1)
                    y = lax.bitcast_convert_type(i, jnp.float32)
                    for _ in range(3):
                        y = y * (1.5 - 0.5 * x * y * y)
                    m = zero + mean_r
                    rs = zero + y
                    for k in range(4):
                        obuf[q, pl.ds(L * k, L)] = (
                            (v[k] - m) * rs * g_regs[k] + b_regs[k])

        start_wb(g)

    wait_wb(NCH - 1)


@jax.jit
def _run(table, idx2d, gamma, beta):
    mesh = plsc.VectorSubcoreMesh(core_axis_name="c", subcore_axis_name="s",
                                  num_cores=NC, num_subcores=NS)
    f = pl.kernel(
        _sc_body,
        out_type=jax.ShapeDtypeStruct((B_CALL, DIM), jnp.float32),
        mesh=mesh,
        scratch_types=[
            pltpu.VMEM((2, SUB, 128), jnp.int32),
            pltpu.VMEM((2, CH, DIM), jnp.float32),
            pltpu.VMEM((CH, DIM), jnp.float32),
            pltpu.VMEM((2, CH, DIM), jnp.float32),
            pltpu.VMEM((DIM,), jnp.float32),
            pltpu.VMEM((DIM,), jnp.float32),
            pltpu.SemaphoreType.DMA,
            pltpu.SemaphoreType.DMA,
        ],
        compiler_params=pltpu.CompilerParams(needs_layout_passes=False,
                                             use_tc_tiling_on_sc=False),
    )
    return f(table, idx2d, gamma, beta)


def kernel(input_ids, table, gamma, beta):
    bsz, seq = input_ids.shape
    idx2d = input_ids.astype(jnp.int32).reshape(-1, 128)
    nrow = idx2d.shape[0] // NSPLIT
    outs = [_run(table, idx2d[i * nrow:(i + 1) * nrow], gamma, beta)
            for i in range(NSPLIT)]
    out = jnp.concatenate(outs, axis=0)
    return out.reshape(bsz, seq, DIM)
